# Initial kernel scaffold; baseline (speedup 1.0000x reference)
import jax
import jax.numpy as jnp
from jax.experimental import pallas as pl


def kernel(x, edge_index, edge_attr, batch, Wn, bn, We, be, c1_W, c1_We, c1_as, c1_ad, c1_ae, c1_b, c2_W, c2_We, c2_as, c2_ad, c2_ae, c2_b, c3_W, c3_We, c3_as, c3_ad, c3_ae, c3_b, lin_W, lin_b):
    raise NotImplementedError("write your pallas kernel here")


# speedup vs baseline: 20.7813x; 20.7813x over previous
import functools

import jax
import jax.numpy as jnp
from jax import lax
from jax.experimental import pallas as pl
from jax.experimental.pallas import tpu as pltpu
from jax.experimental.pallas import tpu_sc as plsc

N = 10000
E = 320000
D = 128
G = 64

NC = 2
NS = 16
NW = NC * NS
L = 16

EW = E // NW
KC = 2000
RB = 80
ZR = 125

_mesh = plsc.VectorSubcoreMesh(core_axis_name="c", subcore_axis_name="s")


@functools.partial(
    pl.kernel,
    out_type=[
        jax.ShapeDtypeStruct((NW * N,), jnp.float32),
        jax.ShapeDtypeStruct((NW * 3 * N,), jnp.float32),
    ],
    mesh=_mesh,
    compiler_params=pltpu.CompilerParams(needs_layout_passes=False),
    scratch_types=[
        pltpu.VMEM((N,), jnp.float32),
        pltpu.VMEM((N,), jnp.float32),
        pltpu.VMEM((N,), jnp.float32),
        pltpu.VMEM((N,), jnp.float32),
        pltpu.VMEM((KC,), jnp.int32),
        pltpu.VMEM((KC,), jnp.float32),
        pltpu.VMEM((KC,), jnp.float32),
        pltpu.VMEM((KC,), jnp.float32),
    ],
)
def _sc_prologue(dst_hbm, ae1_hbm, ae2_hbm, ae3_hbm, degp_out, aesp_out,
                 deg_v, s1_v, s2_v, s3_v, dstc, a1c, a2c, a3c):
    c = lax.axis_index("c")
    s = lax.axis_index("s")
    w = s * NC + c
    base = w * EW
    zf = jnp.zeros((L,), jnp.float32)

    def zb(i, carry):
        deg_v[pl.ds(i * L, L)] = zf
        s1_v[pl.ds(i * L, L)] = zf
        s2_v[pl.ds(i * L, L)] = zf
        s3_v[pl.ds(i * L, L)] = zf
        return carry

    lax.fori_loop(0, N // L, zb, 0)

    ones = jnp.ones((L,), jnp.float32)

    def chunk(k, carry):
        cb = base + k * KC
        pltpu.sync_copy(dst_hbm.at[pl.ds(cb, KC)], dstc)
        pltpu.sync_copy(ae1_hbm.at[pl.ds(cb, KC)], a1c)
        pltpu.sync_copy(ae2_hbm.at[pl.ds(cb, KC)], a2c)
        pltpu.sync_copy(ae3_hbm.at[pl.ds(cb, KC)], a3c)

        def body(j, c2):
            sl = pl.ds(j * L, L)
            idx = dstc[sl]
            plsc.addupdate_scatter(deg_v, [idx], ones)
            plsc.addupdate_scatter(s1_v, [idx], a1c[sl])
            plsc.addupdate_scatter(s2_v, [idx], a2c[sl])
            plsc.addupdate_scatter(s3_v, [idx], a3c[sl])
            return c2

        lax.fori_loop(0, KC // L, body, 0, unroll=2)
        return carry

    lax.fori_loop(0, EW // KC, chunk, 0)
    pltpu.sync_copy(deg_v, degp_out.at[pl.ds(w * N, N)])
    pltpu.sync_copy(s1_v, aesp_out.at[pl.ds((w * 3 + 0) * N, N)])
    pltpu.sync_copy(s2_v, aesp_out.at[pl.ds((w * 3 + 1) * N, N)])
    pltpu.sync_copy(s3_v, aesp_out.at[pl.ds((w * 3 + 2) * N, N)])


@functools.partial(
    pl.kernel,
    out_type=[
        jax.ShapeDtypeStruct((NW * N,), jnp.float32),
        jax.ShapeDtypeStruct((E,), jnp.float32),
    ],
    mesh=_mesh,
    compiler_params=pltpu.CompilerParams(needs_layout_passes=False),
    scratch_types=[
        pltpu.VMEM((N,), jnp.float32),
        pltpu.VMEM((N,), jnp.float32),
        pltpu.VMEM((N,), jnp.float32),
        pltpu.VMEM((KC,), jnp.int32),
        pltpu.VMEM((KC,), jnp.int32),
        pltpu.VMEM((KC,), jnp.float32),
        pltpu.VMEM((KC,), jnp.float32),
    ],
)
def _sc_scalar(src_hbm, dst_hbm, ae_hbm, asv_hbm, adv_hbm,
               denp_out, exq_out,
               asv_v, adv_v, den_v, srcc, dstc, aec, exc):
    c = lax.axis_index("c")
    s = lax.axis_index("s")
    w = s * NC + c
    base = w * EW
    zf = jnp.zeros((L,), jnp.float32)

    pltpu.sync_copy(asv_hbm, asv_v)
    pltpu.sync_copy(adv_hbm, adv_v)

    def zb(i, carry):
        den_v[pl.ds(i * L, L)] = zf
        return carry

    lax.fori_loop(0, N // L, zb, 0)

    def chunk(k, carry):
        cb = base + k * KC
        pltpu.sync_copy(src_hbm.at[pl.ds(cb, KC)], srcc)
        pltpu.sync_copy(dst_hbm.at[pl.ds(cb, KC)], dstc)
        pltpu.sync_copy(ae_hbm.at[pl.ds(cb, KC)], aec)

        def sbody(j, c2):
            sl = pl.ds(j * L, L)
            di = dstc[sl]
            a = plsc.load_gather(asv_v, [srcc[sl]])
            b = plsc.load_gather(adv_v, [di])
            al = a + b + aec[sl]
            al = jnp.where(al >= 0, al, 0.2 * al)
            ex = jnp.exp(al)
            exc[sl] = ex
            plsc.addupdate_scatter(den_v, [di], ex)
            return c2

        lax.fori_loop(0, KC // L, sbody, 0, unroll=2)
        pltpu.sync_copy(exc, exq_out.at[pl.ds(cb, KC)])
        return carry

    lax.fori_loop(0, EW // KC, chunk, 0)
    pltpu.sync_copy(den_v, denp_out.at[pl.ds(w * N, N)])


@functools.partial(
    pl.kernel,
    out_type=[
        jax.ShapeDtypeStruct((NC, N, D), jnp.float32),
    ],
    mesh=_mesh,
    compiler_params=pltpu.CompilerParams(needs_layout_passes=False),
    scratch_types=[
        pltpu.VMEM_SHARED((N, D), jnp.float32),
        pltpu.VMEM((RB,), jnp.int32),
        pltpu.VMEM((RB,), jnp.int32),
        pltpu.VMEM((RB,), jnp.float32),
        pltpu.VMEM((RB, D), jnp.float32),
        pltpu.SemaphoreType.DMA,
    ],
)
def _sc_rows(src_hbm, dst_hbm, exq_hbm, xh_hbm,
             accp_out,
             acc_sh, idxs, idxd, exr, rows, sem):
    c = lax.axis_index("c")
    s = lax.axis_index("s")
    w = s * NC + c
    base = w * EW
    zf = jnp.zeros((L,), jnp.float32)

    def zr(i, carry):
        for q in range(D // L):
            rows[i, pl.ds(q * L, L)] = zf
        return carry

    lax.fori_loop(0, RB, zr, 0)

    def zs(i, carry):
        pltpu.sync_copy(rows, acc_sh.at[pl.ds(s * 624 + i * RB, RB)])
        return carry

    lax.fori_loop(0, 7, zs, 0)
    pltpu.sync_copy(rows.at[pl.ds(0, 64)],
                    acc_sh.at[pl.ds(s * 624 + 560, 64)])

    @pl.when(s == NS - 1)
    def _ztail():
        pltpu.sync_copy(rows.at[pl.ds(0, 16)], acc_sh.at[pl.ds(9984, 16)])

    plsc.subcore_barrier()

    def rbody(r, carry):
        rb = base + r * RB
        pltpu.sync_copy(src_hbm.at[pl.ds(rb, RB)], idxs)
        pltpu.sync_copy(dst_hbm.at[pl.ds(rb, RB)], idxd)
        pltpu.sync_copy(exq_hbm.at[pl.ds(rb, RB)], exr)
        pltpu.async_copy(xh_hbm.at[idxs], rows, sem).wait()

        def scale(rr, c3):
            exv = exr[pl.ds(rr * L, L)]
            for j in range(L):
                wgt = exv[j]
                for q in range(D // L):
                    sl = pl.ds(q * L, L)
                    rows[rr * L + j, sl] = rows[rr * L + j, sl] * wgt
            return c3

        lax.fori_loop(0, RB // L, scale, 0)
        pltpu.sync_copy(rows, acc_sh.at[idxd], add=True)
        return carry

    lax.fori_loop(0, EW // RB, rbody, 0)
    plsc.subcore_barrier()

    def wb(i, carry):
        off = s * 624 + i * RB
        pltpu.sync_copy(acc_sh.at[pl.ds(off, RB)],
                        accp_out.at[c, pl.ds(off, RB)])
        return carry

    lax.fori_loop(0, 7, wb, 0)
    pltpu.sync_copy(acc_sh.at[pl.ds(s * 624 + 560, 64)],
                    accp_out.at[c, pl.ds(s * 624 + 560, 64)])

    @pl.when(s == NS - 1)
    def _wtail():
        pltpu.sync_copy(acc_sh.at[pl.ds(9984, 16)],
                        accp_out.at[c, pl.ds(9984, 16)])


def _lr(v, slope):
    return jnp.where(v >= 0, v, slope * v)


def kernel(x, edge_index, edge_attr, batch, Wn, bn, We, be,
           c1_W, c1_We, c1_as, c1_ad, c1_ae, c1_b,
           c2_W, c2_We, c2_as, c2_ad, c2_ae, c2_b,
           c3_W, c3_We, c3_as, c3_ad, c3_ae, c3_b,
           lin_W, lin_b):
    src_i, dst = edge_index[0], edge_index[1]
    layers = [(c1_W, c1_as[0, 0], c1_ad[0, 0], c1_b),
              (c2_W, c2_as[0, 0], c2_ad[0, 0], c2_b),
              (c3_W, c3_as[0, 0], c3_ad[0, 0], c3_b)]
    V = jnp.stack([c1_We @ c1_ae[0, 0], c2_We @ c2_ae[0, 0],
                   c3_We @ c3_ae[0, 0]], axis=1)
    ae3 = _lr(edge_attr @ We + be, 0.01) @ V
    ae_cols = [jnp.asarray(ae3[:, i], jnp.float32) for i in range(3)]

    degp, aesp = _sc_prologue(dst, *ae_cols)
    deg = jnp.maximum(degp.reshape(NW, N).sum(0), 1.0)
    la3 = aesp.reshape(NW, 3, N).sum(0) / deg[None, :]

    h = _lr(x @ Wn + bn, 0.01)
    for l, (W, asw, adw, b) in enumerate(layers):
        xh = h @ W
        asv = xh @ asw
        adv = xh @ adw
        denp, exq = _sc_scalar(src_i, dst, ae_cols[l], asv, adv)
        accp, = _sc_rows(src_i, dst, exq, xh)
        exl = jnp.exp(_lr(asv + adv + la3[l], 0.2))
        denom = denp.reshape(NW, N).sum(0) + exl
        acc = accp.sum(0) + exl[:, None] * xh
        h = _lr(acc / (denom[:, None] + 1e-16) + b, 0.01)

    pooled = jax.ops.segment_sum(h, batch, num_segments=G)
    return pooled @ lin_W + lin_b


# speedup vs baseline: 39.6540x; 1.9082x over previous
import functools

import jax
import jax.numpy as jnp
from jax import lax
from jax.experimental import pallas as pl
from jax.experimental.pallas import tpu as pltpu
from jax.experimental.pallas import tpu_sc as plsc

N = 10000
E = 320000
D = 128
G = 64

NC = 2
NS = 16
NW = NC * NS
L = 16

EW = E // NW
KC = 2000
RB = 80
ZR = 125

_mesh = plsc.VectorSubcoreMesh(core_axis_name="c", subcore_axis_name="s")


@functools.partial(
    pl.kernel,
    out_type=[
        jax.ShapeDtypeStruct((NW * N,), jnp.float32),
        jax.ShapeDtypeStruct((NW * 3 * N,), jnp.float32),
    ],
    mesh=_mesh,
    compiler_params=pltpu.CompilerParams(needs_layout_passes=False),
    scratch_types=[
        pltpu.VMEM((N,), jnp.float32),
        pltpu.VMEM((N,), jnp.float32),
        pltpu.VMEM((N,), jnp.float32),
        pltpu.VMEM((N,), jnp.float32),
        pltpu.VMEM((KC,), jnp.int32),
        pltpu.VMEM((KC,), jnp.float32),
        pltpu.VMEM((KC,), jnp.float32),
        pltpu.VMEM((KC,), jnp.float32),
    ],
)
def _sc_prologue(dst_hbm, ae1_hbm, ae2_hbm, ae3_hbm, degp_out, aesp_out,
                 deg_v, s1_v, s2_v, s3_v, dstc, a1c, a2c, a3c):
    c = lax.axis_index("c")
    s = lax.axis_index("s")
    w = s * NC + c
    base = w * EW
    zf = jnp.zeros((L,), jnp.float32)

    def zb(i, carry):
        deg_v[pl.ds(i * L, L)] = zf
        s1_v[pl.ds(i * L, L)] = zf
        s2_v[pl.ds(i * L, L)] = zf
        s3_v[pl.ds(i * L, L)] = zf
        return carry

    lax.fori_loop(0, N // L, zb, 0)

    ones = jnp.ones((L,), jnp.float32)

    def chunk(k, carry):
        cb = base + k * KC
        pltpu.sync_copy(dst_hbm.at[pl.ds(cb, KC)], dstc)
        pltpu.sync_copy(ae1_hbm.at[pl.ds(cb, KC)], a1c)
        pltpu.sync_copy(ae2_hbm.at[pl.ds(cb, KC)], a2c)
        pltpu.sync_copy(ae3_hbm.at[pl.ds(cb, KC)], a3c)

        def body(j, c2):
            sl = pl.ds(j * L, L)
            idx = dstc[sl]
            plsc.addupdate_scatter(deg_v, [idx], ones)
            plsc.addupdate_scatter(s1_v, [idx], a1c[sl])
            plsc.addupdate_scatter(s2_v, [idx], a2c[sl])
            plsc.addupdate_scatter(s3_v, [idx], a3c[sl])
            return c2

        lax.fori_loop(0, KC // L, body, 0, unroll=2)
        return carry

    lax.fori_loop(0, EW // KC, chunk, 0)
    pltpu.sync_copy(deg_v, degp_out.at[pl.ds(w * N, N)])
    pltpu.sync_copy(s1_v, aesp_out.at[pl.ds((w * 3 + 0) * N, N)])
    pltpu.sync_copy(s2_v, aesp_out.at[pl.ds((w * 3 + 1) * N, N)])
    pltpu.sync_copy(s3_v, aesp_out.at[pl.ds((w * 3 + 2) * N, N)])


@functools.partial(
    pl.kernel,
    out_type=[
        jax.ShapeDtypeStruct((NW * N,), jnp.float32),
        jax.ShapeDtypeStruct((E,), jnp.float32),
    ],
    mesh=_mesh,
    compiler_params=pltpu.CompilerParams(needs_layout_passes=False),
    scratch_types=[
        pltpu.VMEM((N,), jnp.float32),
        pltpu.VMEM((N,), jnp.float32),
        pltpu.VMEM((N,), jnp.float32),
        pltpu.VMEM((KC,), jnp.int32),
        pltpu.VMEM((KC,), jnp.int32),
        pltpu.VMEM((KC,), jnp.float32),
        pltpu.VMEM((KC,), jnp.float32),
    ],
)
def _sc_scalar(src_hbm, dst_hbm, ae_hbm, asv_hbm, adv_hbm,
               denp_out, exq_out,
               asv_v, adv_v, den_v, srcc, dstc, aec, exc):
    c = lax.axis_index("c")
    s = lax.axis_index("s")
    w = s * NC + c
    base = w * EW
    zf = jnp.zeros((L,), jnp.float32)

    pltpu.sync_copy(asv_hbm, asv_v)
    pltpu.sync_copy(adv_hbm, adv_v)

    def zb(i, carry):
        den_v[pl.ds(i * L, L)] = zf
        return carry

    lax.fori_loop(0, N // L, zb, 0)

    def chunk(k, carry):
        cb = base + k * KC
        pltpu.sync_copy(src_hbm.at[pl.ds(cb, KC)], srcc)
        pltpu.sync_copy(dst_hbm.at[pl.ds(cb, KC)], dstc)
        pltpu.sync_copy(ae_hbm.at[pl.ds(cb, KC)], aec)

        def sbody(j, c2):
            sl = pl.ds(j * L, L)
            di = dstc[sl]
            a = plsc.load_gather(asv_v, [srcc[sl]])
            b = plsc.load_gather(adv_v, [di])
            al = a + b + aec[sl]
            al = jnp.where(al >= 0, al, 0.2 * al)
            ex = jnp.exp(al)
            exc[sl] = ex
            plsc.addupdate_scatter(den_v, [di], ex)
            return c2

        lax.fori_loop(0, KC // L, sbody, 0, unroll=2)
        pltpu.sync_copy(exc, exq_out.at[pl.ds(cb, KC)])
        return carry

    lax.fori_loop(0, EW // KC, chunk, 0)
    pltpu.sync_copy(den_v, denp_out.at[pl.ds(w * N, N)])


NBAT = KC // RB


@functools.partial(
    pl.kernel,
    out_type=[
        jax.ShapeDtypeStruct((NC, N, D), jnp.float32),
    ],
    mesh=_mesh,
    compiler_params=pltpu.CompilerParams(needs_layout_passes=False),
    scratch_types=[
        pltpu.VMEM_SHARED((N, D), jnp.float32),
        pltpu.VMEM((KC,), jnp.int32),
        pltpu.VMEM((KC,), jnp.int32),
        pltpu.VMEM((KC,), jnp.float32),
        pltpu.VMEM((RB,), jnp.int32),
        pltpu.VMEM((RB,), jnp.int32),
        pltpu.VMEM((RB, D), jnp.float32),
        pltpu.VMEM((RB, D), jnp.float32),
        pltpu.SemaphoreType.DMA,
        pltpu.SemaphoreType.DMA,
    ],
)
def _sc_rows(src_hbm, dst_hbm, exq_hbm, xh_hbm,
             accp_out,
             acc_sh, srcc, dstc, exc, idxd0, idxd1, rows0, rows1,
             sem0, sem1):
    c = lax.axis_index("c")
    s = lax.axis_index("s")
    w = s * NC + c
    base = w * EW
    zf = jnp.zeros((L,), jnp.float32)
    bufs = ((idxd0, rows0, sem0), (idxd1, rows1, sem1))

    def zr(i, carry):
        for q in range(D // L):
            rows0[i, pl.ds(q * L, L)] = zf
        return carry

    lax.fori_loop(0, RB, zr, 0)

    def zs(i, carry):
        pltpu.sync_copy(rows0, acc_sh.at[pl.ds(s * 624 + i * RB, RB)])
        return carry

    lax.fori_loop(0, 7, zs, 0)
    pltpu.sync_copy(rows0.at[pl.ds(0, 64)],
                    acc_sh.at[pl.ds(s * 624 + 560, 64)])

    @pl.when(s == NS - 1)
    def _ztail():
        pltpu.sync_copy(rows0.at[pl.ds(0, 16)], acc_sh.at[pl.ds(9984, 16)])

    plsc.subcore_barrier()

    def _issue(r, b):
        _, rows, sem = bufs[b]
        pltpu.async_copy(xh_hbm.at[srcc.at[pl.ds(r * RB, RB)]], rows, sem)

    def _finish(r, b):
        idxd, rows, sem = bufs[b]
        pltpu.make_async_copy(
            xh_hbm.at[srcc.at[pl.ds(r * RB, RB)]], rows, sem).wait()

        def cp(i, carry):
            sl = pl.ds(i * L, L)
            idxd[sl] = dstc[pl.ds(r * RB + i * L, L)]
            return carry

        lax.fori_loop(0, RB // L, cp, 0)

        def scale(rr, carry):
            exv = exc[pl.ds(r * RB + rr * L, L)]
            for j in range(L):
                wgt = exv[j]
                for q in range(D // L):
                    sl = pl.ds(q * L, L)
                    rows[rr * L + j, sl] = rows[rr * L + j, sl] * wgt
            return carry

        lax.fori_loop(0, RB // L, scale, 0)
        pltpu.sync_copy(rows, acc_sh.at[idxd], add=True)

    def chunk(k, carry):
        cb = base + k * KC
        pltpu.sync_copy(src_hbm.at[pl.ds(cb, KC)], srcc)
        pltpu.sync_copy(dst_hbm.at[pl.ds(cb, KC)], dstc)
        pltpu.sync_copy(exq_hbm.at[pl.ds(cb, KC)], exc)
        _issue(0, 0)

        def pair(r2, c2):
            r0 = 2 * r2

            @pl.when(r0 + 1 < NBAT)
            def _i1():
                _issue(r0 + 1, 1)

            _finish(r0, 0)

            @pl.when(r0 + 2 < NBAT)
            def _i0():
                _issue(r0 + 2, 0)

            @pl.when(r0 + 1 < NBAT)
            def _f1():
                _finish(r0 + 1, 1)

            return c2

        lax.fori_loop(0, (NBAT + 1) // 2, pair, 0)
        return carry

    lax.fori_loop(0, EW // KC, chunk, 0)
    plsc.subcore_barrier()

    def wb(i, carry):
        off = s * 624 + i * RB
        pltpu.sync_copy(acc_sh.at[pl.ds(off, RB)],
                        accp_out.at[c, pl.ds(off, RB)])
        return carry

    lax.fori_loop(0, 7, wb, 0)
    pltpu.sync_copy(acc_sh.at[pl.ds(s * 624 + 560, 64)],
                    accp_out.at[c, pl.ds(s * 624 + 560, 64)])

    @pl.when(s == NS - 1)
    def _wtail():
        pltpu.sync_copy(acc_sh.at[pl.ds(9984, 16)],
                        accp_out.at[c, pl.ds(9984, 16)])


def _lr(v, slope):
    return jnp.where(v >= 0, v, slope * v)


def kernel(x, edge_index, edge_attr, batch, Wn, bn, We, be,
           c1_W, c1_We, c1_as, c1_ad, c1_ae, c1_b,
           c2_W, c2_We, c2_as, c2_ad, c2_ae, c2_b,
           c3_W, c3_We, c3_as, c3_ad, c3_ae, c3_b,
           lin_W, lin_b):
    src_i, dst = edge_index[0], edge_index[1]
    layers = [(c1_W, c1_as[0, 0], c1_ad[0, 0], c1_b),
              (c2_W, c2_as[0, 0], c2_ad[0, 0], c2_b),
              (c3_W, c3_as[0, 0], c3_ad[0, 0], c3_b)]
    V = jnp.stack([c1_We @ c1_ae[0, 0], c2_We @ c2_ae[0, 0],
                   c3_We @ c3_ae[0, 0]], axis=1)
    ae3 = _lr(edge_attr @ We + be, 0.01) @ V
    ae_cols = [jnp.asarray(ae3[:, i], jnp.float32) for i in range(3)]

    degp, aesp = _sc_prologue(dst, *ae_cols)
    deg = jnp.maximum(degp.reshape(NW, N).sum(0), 1.0)
    la3 = aesp.reshape(NW, 3, N).sum(0) / deg[None, :]

    h = _lr(x @ Wn + bn, 0.01)
    for l, (W, asw, adw, b) in enumerate(layers):
        xh = h @ W
        asv = xh @ asw
        adv = xh @ adw
        denp, exq = _sc_scalar(src_i, dst, ae_cols[l], asv, adv)
        accp, = _sc_rows(src_i, dst, exq, xh)
        exl = jnp.exp(_lr(asv + adv + la3[l], 0.2))
        denom = denp.reshape(NW, N).sum(0) + exl
        acc = accp.sum(0) + exl[:, None] * xh
        h = _lr(acc / (denom[:, None] + 1e-16) + b, 0.01)

    pooled = jax.ops.segment_sum(h, batch, num_segments=G)
    return pooled @ lin_W + lin_b
